# baseline (speedup 1.0000x reference)
import functools
import jax
import jax.numpy as jnp
from jax import lax
from jax.experimental import pallas as pl
from jax.experimental.pallas import tpu as pltpu, tpu_sc as plsc

GRID = 32
D_MODEL = 1024
HALF = D_MODEL // 2
N_HCHUNK = 16
N_BGROUP = 2
H_PER = GRID // N_HCHUNK
FIRE = 16


def _sc_body(n_batch, row_hbm, col_hbm, out_hbm, chunk, sem):
    wid = lax.axis_index("s") * 2 + lax.axis_index("c")
    hc = wid % N_HCHUNK
    bg = wid // N_HCHUNK
    nb = n_batch // N_BGROUP
    rows = H_PER * GRID
    stage = []
    for sub in range(H_PER):
        stage.append(
            pltpu.async_copy(
                col_hbm, chunk.at[pl.ds(sub * GRID, GRID), pl.ds(0, HALF)], sem
            )
        )
        stage += [
            pltpu.async_copy(
                row_hbm.at[hc * H_PER + sub],
                chunk.at[sub * GRID + w, pl.ds(HALF, HALF)],
                sem,
            )
            for w in range(GRID)
        ]
    for c in stage:
        c.wait()
    for g in range(0, nb, FIRE):
        copies = [
            pltpu.async_copy(
                chunk,
                out_hbm.at[bg * nb + b, pl.ds(hc * rows, rows), :],
                sem,
            )
            for b in range(g, min(g + FIRE, nb))
        ]
        for c in copies:
            c.wait()


def kernel(x, row_embed, col_embed):
    b = x.shape[0]
    mesh = plsc.VectorSubcoreMesh(core_axis_name="c", subcore_axis_name="s")
    run = functools.partial(
        pl.kernel,
        out_type=jax.ShapeDtypeStruct((b, GRID * GRID, D_MODEL), jnp.float32),
        mesh=mesh,
        scratch_types=[
            pltpu.VMEM((H_PER * GRID, D_MODEL), jnp.float32),
            pltpu.SemaphoreType.DMA,
        ],
    )(functools.partial(_sc_body, b))
    return run(row_embed, col_embed)


# speedup vs baseline: 1.1351x; 1.1351x over previous
import functools
import jax
import jax.numpy as jnp
from jax import lax
from jax.experimental import pallas as pl
from jax.experimental.pallas import tpu as pltpu, tpu_sc as plsc

GRID = 32
D_MODEL = 1024
HALF = D_MODEL // 2
FIRE = 8


def _sc_body(n_batch, row_hbm, col_hbm, out_hbm, chunk, sem):
    wid = lax.axis_index("s") * 2 + lax.axis_index("c")
    stage = [pltpu.async_copy(col_hbm, chunk.at[:, pl.ds(0, HALF)], sem)]
    stage += [
        pltpu.async_copy(row_hbm.at[wid], chunk.at[w, pl.ds(HALF, HALF)], sem)
        for w in range(GRID)
    ]
    for c in stage:
        c.wait()
    for g in range(0, n_batch, FIRE):
        copies = [
            pltpu.async_copy(chunk, out_hbm.at[b, pl.ds(wid * GRID, GRID), :], sem)
            for b in range(g, min(g + FIRE, n_batch))
        ]
        for c in copies:
            c.wait()


def kernel(x, row_embed, col_embed):
    b = x.shape[0]
    mesh = plsc.VectorSubcoreMesh(core_axis_name="c", subcore_axis_name="s")
    run = functools.partial(
        pl.kernel,
        out_type=jax.ShapeDtypeStruct((b, GRID * GRID, D_MODEL), jnp.float32),
        mesh=mesh,
        scratch_types=[
            pltpu.VMEM((GRID, D_MODEL), jnp.float32),
            pltpu.SemaphoreType.DMA,
        ],
    )(functools.partial(_sc_body, b))
    return run(row_embed, col_embed)


# speedup vs baseline: 1.1643x; 1.0257x over previous
import functools
import jax
import jax.numpy as jnp
from jax import lax
from jax.experimental import pallas as pl
from jax.experimental.pallas import tpu as pltpu, tpu_sc as plsc

GRID = 32
D_MODEL = 1024
HALF = D_MODEL // 2
FIRE = 8


def _sc_body(n_batch, row_hbm, col_hbm, out_hbm, chunk, sem):
    wid = lax.axis_index("s") * 2 + lax.axis_index("c")
    stage = [pltpu.async_copy(col_hbm, chunk.at[:, pl.ds(0, HALF)], sem)]
    stage += [
        pltpu.async_copy(row_hbm.at[wid], chunk.at[w, pl.ds(HALF, HALF)], sem)
        for w in range(GRID)
    ]
    for c in stage:
        c.wait()
    for g in range(0, n_batch, FIRE):
        copies = [
            pltpu.async_copy(
                chunk,
                out_hbm.at[
                    lax.rem(b + 2 * wid, n_batch), pl.ds(wid * GRID, GRID), :
                ],
                sem,
            )
            for b in range(g, min(g + FIRE, n_batch))
        ]
        for c in copies:
            c.wait()


def kernel(x, row_embed, col_embed):
    b = x.shape[0]
    mesh = plsc.VectorSubcoreMesh(core_axis_name="c", subcore_axis_name="s")
    run = functools.partial(
        pl.kernel,
        out_type=jax.ShapeDtypeStruct((b, GRID * GRID, D_MODEL), jnp.float32),
        mesh=mesh,
        scratch_types=[
            pltpu.VMEM((GRID, D_MODEL), jnp.float32),
            pltpu.SemaphoreType.DMA,
        ],
    )(functools.partial(_sc_body, b))
    return run(row_embed, col_embed)
